# baseline (speedup 1.0000x reference)
import jax
import jax.numpy as jnp
from jax.experimental import pallas as pl
from jax.experimental.pallas import tpu as pltpu

E, N, D = 8, 4096, 1024
TN = 256


def _moe_body(x_ref, wr_ref, br_ref, we_hbm, be_ref, out_ref, we_vmem, sems):
    nt = pl.program_id(0)

    @pl.when(nt == 0)
    def _start_weight_copies():
        for e in range(E):
            pltpu.make_async_copy(
                we_hbm.at[e], we_vmem.at[e], sems.at[e]
            ).start()

    x0 = x_ref[0]
    logits = (
        jnp.dot(x0, wr_ref[...], preferred_element_type=jnp.float32)
        + br_ref[...]
    )
    m = jnp.max(logits, axis=-1, keepdims=True)
    ex = jnp.exp(logits - m)
    gates = ex / jnp.sum(ex, axis=-1, keepdims=True)
    gn = gates / jnp.sum(gates, axis=-1, keepdims=True)

    acc = jnp.dot(gn, be_ref[...], preferred_element_type=jnp.float32)
    for e in range(E):
        @pl.when(nt == 0)
        def _wait_weight_copy(e=e):
            pltpu.make_async_copy(
                we_hbm.at[e], we_vmem.at[e], sems.at[e]
            ).wait()

        partial = jnp.dot(
            x_ref[e], we_vmem[e], preferred_element_type=jnp.float32
        )
        acc = acc + gn[:, e : e + 1] * partial
    out_ref[...] = acc


@jax.jit
def _moe(x, Wr, br, We, be):
    num_tiles = N // TN
    return pl.pallas_call(
        _moe_body,
        grid=(num_tiles,),
        in_specs=[
            pl.BlockSpec((E, TN, D), lambda nt: (0, nt, 0)),
            pl.BlockSpec((D, E), lambda nt: (0, 0)),
            pl.BlockSpec((1, E), lambda nt: (0, 0)),
            pl.BlockSpec(memory_space=pltpu.MemorySpace.HBM),
            pl.BlockSpec((E, D), lambda nt: (0, 0)),
        ],
        out_specs=pl.BlockSpec((TN, D), lambda nt: (nt, 0)),
        out_shape=jax.ShapeDtypeStruct((N, D), jnp.float32),
        scratch_shapes=[
            pltpu.VMEM((E, D, D), jnp.float32),
            pltpu.SemaphoreType.DMA((E,)),
        ],
        compiler_params=pltpu.CompilerParams(
            dimension_semantics=("parallel",),
        ),
    )(x, Wr, br, We, be)


def kernel(x, Wr, br, We, be):
    return _moe(x, Wr, br.reshape(1, E), We, be)


# speedup vs baseline: 1.1912x; 1.1912x over previous
import jax
import jax.numpy as jnp
from jax.experimental import pallas as pl
from jax.experimental.pallas import tpu as pltpu

E, N, D = 8, 4096, 1024
TN = 256


def _moe_body(x_ref, wr_ref, br_ref, we_hbm, be_ref, out_ref, we_vmem, sems):
    nt = pl.program_id(0)

    @pl.when(nt == 0)
    def _start_weight_copies():
        for e in range(E):
            pltpu.make_async_copy(
                we_hbm.at[e], we_vmem.at[e], sems.at[e]
            ).start()

    x0 = x_ref[0]
    logits = (
        jnp.dot(x0, wr_ref[...], preferred_element_type=jnp.float32)
        + br_ref[...]
    )
    m = jnp.max(logits, axis=-1, keepdims=True)
    ex = jnp.exp(logits - m)
    gates = ex / jnp.sum(ex, axis=-1, keepdims=True)
    gn = gates / jnp.sum(gates, axis=-1, keepdims=True)

    @pl.when(nt == 0)
    def _wait_weight_copies():
        for e in range(E):
            pltpu.make_async_copy(
                we_hbm.at[e], we_vmem.at[e], sems.at[e]
            ).wait()

    acc = jnp.dot(gn, be_ref[...], preferred_element_type=jnp.float32)
    for e in range(E):
        partial = jnp.dot(
            x_ref[e], we_vmem[e], preferred_element_type=jnp.float32
        )
        acc = acc + gn[:, e : e + 1] * partial
    out_ref[...] = acc


@jax.jit
def _moe(x, Wr, br, We, be):
    num_tiles = N // TN
    return pl.pallas_call(
        _moe_body,
        grid=(num_tiles,),
        in_specs=[
            pl.BlockSpec((E, TN, D), lambda nt: (0, nt, 0)),
            pl.BlockSpec((D, E), lambda nt: (0, 0)),
            pl.BlockSpec((1, E), lambda nt: (0, 0)),
            pl.BlockSpec(memory_space=pltpu.MemorySpace.HBM),
            pl.BlockSpec((E, D), lambda nt: (0, 0)),
        ],
        out_specs=pl.BlockSpec((TN, D), lambda nt: (nt, 0)),
        out_shape=jax.ShapeDtypeStruct((N, D), jnp.float32),
        scratch_shapes=[
            pltpu.VMEM((E, D, D), jnp.float32),
            pltpu.SemaphoreType.DMA((E,)),
        ],
        compiler_params=pltpu.CompilerParams(
            dimension_semantics=("parallel",),
        ),
    )(x, Wr, br, We, be)


def kernel(x, Wr, br, We, be):
    return _moe(x, Wr, br.reshape(1, E), We, be)


# speedup vs baseline: 1.2078x; 1.0139x over previous
import jax
import jax.numpy as jnp
from jax.experimental import pallas as pl
from jax.experimental.pallas import tpu as pltpu

E, N, D = 8, 4096, 1024
TN = 512
EG = 2
EPG = E // EG


def _moe_body(x_ref, wr_ref, br_ref, we_ref, be_ref, out_ref, gn_ref):
    eg = pl.program_id(1)

    @pl.when(eg == 0)
    def _init():
        x0 = x_ref[0]
        logits = (
            jnp.dot(x0, wr_ref[...], preferred_element_type=jnp.float32)
            + br_ref[...]
        )
        m = jnp.max(logits, axis=-1, keepdims=True)
        ex = jnp.exp(logits - m)
        gates = ex / jnp.sum(ex, axis=-1, keepdims=True)
        gn = gates / jnp.sum(gates, axis=-1, keepdims=True)
        gn_ref[...] = gn
        out_ref[...] = jnp.dot(gn, be_ref[...], preferred_element_type=jnp.float32)

    gn = gn_ref[...]
    lane = jax.lax.broadcasted_iota(jnp.int32, (1, E), 1)
    acc = out_ref[...]
    for e in range(EPG):
        idx = eg * EPG + e
        gcol = jnp.sum(
            gn * (lane == idx).astype(jnp.float32), axis=-1, keepdims=True
        )
        partial = jnp.dot(
            x_ref[e], we_ref[idx], preferred_element_type=jnp.float32
        )
        acc = acc + gcol * partial
    out_ref[...] = acc


@jax.jit
def _moe(x, Wr, br, We, be):
    num_tiles = N // TN
    return pl.pallas_call(
        _moe_body,
        grid=(num_tiles, EG),
        in_specs=[
            pl.BlockSpec((EPG, TN, D), lambda nt, eg: (eg, nt, 0)),
            pl.BlockSpec((D, E), lambda nt, eg: (0, 0)),
            pl.BlockSpec((1, E), lambda nt, eg: (0, 0)),
            pl.BlockSpec((E, D, D), lambda nt, eg: (0, 0, 0)),
            pl.BlockSpec((E, D), lambda nt, eg: (0, 0)),
        ],
        out_specs=pl.BlockSpec((TN, D), lambda nt, eg: (nt, 0)),
        out_shape=jax.ShapeDtypeStruct((N, D), jnp.float32),
        scratch_shapes=[pltpu.VMEM((TN, E), jnp.float32)],
        compiler_params=pltpu.CompilerParams(
            dimension_semantics=("parallel", "arbitrary"),
        ),
    )(x, Wr, br, We, be)


def kernel(x, Wr, br, We, be):
    return _moe(x, Wr, br.reshape(1, E), We, be)


# speedup vs baseline: 1.3447x; 1.1134x over previous
import jax
import jax.numpy as jnp
from jax.experimental import pallas as pl
from jax.experimental.pallas import tpu as pltpu

E, N, D = 8, 4096, 1024
TN = 256


def _moe_body(x_ref, wr_ref, br_ref, we_ref, be_ref, out_ref):
    x0 = x_ref[0]
    logits = (
        jnp.dot(x0, wr_ref[...], preferred_element_type=jnp.float32)
        + br_ref[...]
    )
    m = jnp.max(logits, axis=-1, keepdims=True)
    ex = jnp.exp(logits - m)
    gates = ex / jnp.sum(ex, axis=-1, keepdims=True)
    gn = gates / jnp.sum(gates, axis=-1, keepdims=True)

    acc = jnp.dot(gn, be_ref[...], preferred_element_type=jnp.float32)
    for e in range(E):
        partial = jnp.dot(
            x_ref[e], we_ref[e], preferred_element_type=jnp.float32
        )
        acc = acc + gn[:, e : e + 1] * partial
    out_ref[...] = acc


@jax.jit
def _moe(x, Wr, br, We, be):
    num_tiles = N // TN
    return pl.pallas_call(
        _moe_body,
        grid=(num_tiles,),
        in_specs=[
            pl.BlockSpec((E, TN, D), lambda nt: (0, nt, 0)),
            pl.BlockSpec((D, E), lambda nt: (0, 0)),
            pl.BlockSpec((1, E), lambda nt: (0, 0)),
            pl.BlockSpec((E, D, D), lambda nt: (0, 0, 0)),
            pl.BlockSpec((E, D), lambda nt: (0, 0)),
        ],
        out_specs=pl.BlockSpec((TN, D), lambda nt: (nt, 0)),
        out_shape=jax.ShapeDtypeStruct((N, D), jnp.float32),
        compiler_params=pltpu.CompilerParams(
            dimension_semantics=("parallel",),
        ),
    )(x, Wr, br, We, be)


def kernel(x, Wr, br, We, be):
    return _moe(x, Wr, br.reshape(1, E), We, be)
